# baseline (speedup 1.0000x reference)
import functools
import math

import jax
import jax.numpy as jnp
from jax.experimental import pallas as pl
from jax.experimental.pallas import tpu as pltpu

_K = 64


def _attn_block_kernel(x_ref, o_ref, *, scale):
    xb = x_ref[...]
    s = jax.lax.dot_general(
        xb, xb, (((2,), (2,)), ((0,), (0,))),
        preferred_element_type=jnp.float32) * scale
    n = xb.shape[1]
    row = jax.lax.broadcasted_iota(jnp.int32, (1, n, n), 1)
    col = jax.lax.broadcasted_iota(jnp.int32, (1, n, n), 2)
    s = jnp.where(col <= row, s, -jnp.inf)
    m = jnp.max(s, axis=2, keepdims=True)
    p = jnp.exp(s - m)
    z = jnp.sum(p, axis=2, keepdims=True)
    p = p / z
    o_ref[...] = jax.lax.dot_general(
        p, xb, (((2,), (1,)), ((0,), (0,))),
        preferred_element_type=jnp.float32)


def kernel(x):
    b, t, e = x.shape
    scale = 1.0 / math.sqrt(e)
    nblocks = t // _K
    bp = 32
    grid = (nblocks // bp,)
    x2 = x.reshape(nblocks, _K, e)
    out = pl.pallas_call(
        functools.partial(_attn_block_kernel, scale=scale),
        grid=grid,
        in_specs=[pl.BlockSpec((bp, _K, e), lambda i: (i, 0, 0))],
        out_specs=pl.BlockSpec((bp, _K, e), lambda i: (i, 0, 0)),
        out_shape=jax.ShapeDtypeStruct((nblocks, _K, e), jnp.float32),
        compiler_params=pltpu.CompilerParams(
            dimension_semantics=("arbitrary",),
        ),
    )(x2)
    return out.reshape(b, t, e)


# speedup vs baseline: 1.0405x; 1.0405x over previous
import functools
import math

import jax
import jax.numpy as jnp
from jax.experimental import pallas as pl
from jax.experimental.pallas import tpu as pltpu

_K = 64


def _attn_block_kernel(x_ref, o_ref, *, scale):
    xb = x_ref[...]
    s = jax.lax.dot_general(
        xb, xb, (((2,), (2,)), ((0,), (0,))),
        preferred_element_type=jnp.float32) * scale
    n = xb.shape[1]
    row = jax.lax.broadcasted_iota(jnp.int32, (1, n, n), 1)
    col = jax.lax.broadcasted_iota(jnp.int32, (1, n, n), 2)
    s = jnp.where(col <= row, s, -jnp.inf)
    m = jnp.max(s, axis=2, keepdims=True)
    p = jnp.exp(s - m)
    z = jnp.sum(p, axis=2, keepdims=True)
    p = p / z
    o_ref[...] = jax.lax.dot_general(
        p, xb, (((2,), (1,)), ((0,), (0,))),
        preferred_element_type=jnp.float32)


def kernel(x):
    b, t, e = x.shape
    scale = 1.0 / math.sqrt(e)
    nblocks = t // _K
    bp = 8
    grid = (nblocks // bp,)
    x2 = x.reshape(nblocks, _K, e)
    out = pl.pallas_call(
        functools.partial(_attn_block_kernel, scale=scale),
        grid=grid,
        in_specs=[pl.BlockSpec((bp, _K, e), lambda i: (i, 0, 0))],
        out_specs=pl.BlockSpec((bp, _K, e), lambda i: (i, 0, 0)),
        out_shape=jax.ShapeDtypeStruct((nblocks, _K, e), jnp.float32),
        compiler_params=pltpu.CompilerParams(
            dimension_semantics=("parallel",),
        ),
    )(x2)
    return out.reshape(b, t, e)


# speedup vs baseline: 1.2987x; 1.2482x over previous
import functools
import math

import jax
import jax.numpy as jnp
from jax.experimental import pallas as pl
from jax.experimental.pallas import tpu as pltpu

_K = 64


def _attn_block_kernel(x_ref, o_ref, *, scale):
    xb = x_ref[...]
    s = jax.lax.dot_general(
        xb, xb, (((2,), (2,)), ((0,), (0,))),
        preferred_element_type=jnp.float32) * scale
    n = xb.shape[1]
    row = jax.lax.broadcasted_iota(jnp.int32, (1, n, n), 1)
    col = jax.lax.broadcasted_iota(jnp.int32, (1, n, n), 2)
    s = jnp.where(col <= row, s, -jnp.inf)
    m = jnp.max(s, axis=2, keepdims=True)
    p = jnp.exp(s - m)
    z = jnp.sum(p, axis=2, keepdims=True)
    p = p / z
    o_ref[...] = jax.lax.dot_general(
        p, xb, (((2,), (1,)), ((0,), (0,))),
        preferred_element_type=jnp.float32)


def kernel(x):
    b, t, e = x.shape
    scale = 1.0 / math.sqrt(e)
    nblocks = t // _K
    bp = 16
    grid = (nblocks // bp,)
    x2 = x.reshape(nblocks, _K, e)
    out = pl.pallas_call(
        functools.partial(_attn_block_kernel, scale=scale),
        grid=grid,
        in_specs=[pl.BlockSpec((bp, _K, e), lambda i: (i, 0, 0))],
        out_specs=pl.BlockSpec((bp, _K, e), lambda i: (i, 0, 0)),
        out_shape=jax.ShapeDtypeStruct((nblocks, _K, e), jnp.float32),
        compiler_params=pltpu.CompilerParams(
            dimension_semantics=("parallel",),
        ),
    )(x2)
    return out.reshape(b, t, e)
